# Initial kernel scaffold; baseline (speedup 1.0000x reference)
import jax
import jax.numpy as jnp
from jax.experimental import pallas as pl


def kernel(x, edge_index, batch, W_gat, att_src, att_dst, b_gat, W1, b1, W2, b2, Wf, bf):
    raise NotImplementedError("write your pallas kernel here")


# speedup vs baseline: 8.9350x; 8.9350x over previous
import functools

import jax
import jax.numpy as jnp
from jax import lax
from jax.experimental import pallas as pl
from jax.experimental.pallas import tpu as pltpu
from jax.experimental.pallas import tpu_sc as plsc

NN = 10000
EE = 160000
DIN = 128
NH = 4
CH = 128
NG = 64

NC = 2
NS = 16
NW = NC * NS

EG = EE + NN
EGP = 172032
TG = EGP // NW
G1C = 128
G1N = TG // G1C
S2C = 128
S2N = TG // S2C

ENP = 163840
TN = ENP // NW
S3C = 128
S3N = TN // S3C

SR = 10240
ZB = 640
CB = 624
CBT = 16

BN = 1000
BE = 1024


def _tc1_body(x_ref, w_ref, atts_ref, attd_ref, haug_ref, ad_ref):
    h = jnp.dot(x_ref[...], w_ref[...], preferred_element_type=jnp.float32)
    hr = h.reshape(BN, NH, CH)
    a_s = jnp.sum(hr * atts_ref[...][None], axis=-1)
    a_d = jnp.sum(hr * attd_ref[...][None], axis=-1)
    z = jnp.zeros((BN, 124), jnp.float32)
    haug_ref[...] = jnp.concatenate([h, a_s, z], axis=1)
    ad_ref[...] = jnp.concatenate([a_d, z], axis=1)


def _tc1(x, w_gat, att_src, att_dst):
    return pl.pallas_call(
        _tc1_body,
        grid=(NN // BN,),
        in_specs=[
            pl.BlockSpec((BN, DIN), lambda i: (i, 0)),
            pl.BlockSpec((DIN, NH * CH), lambda i: (0, 0)),
            pl.BlockSpec((NH, CH), lambda i: (0, 0)),
            pl.BlockSpec((NH, CH), lambda i: (0, 0)),
        ],
        out_specs=[
            pl.BlockSpec((BN, NH * CH + CH), lambda i: (i, 0)),
            pl.BlockSpec((BN, CH), lambda i: (i, 0)),
        ],
        out_shape=[
            jax.ShapeDtypeStruct((NN, NH * CH + CH), jnp.float32),
            jax.ShapeDtypeStruct((NN, CH), jnp.float32),
        ],
    )(x, w_gat, att_src, att_dst)


def _sc1_body(haug_hbm, ad_hbm, srcg_hbm, dstg_hbm,
              hsa_hbm, adg_hbm,
              idxs_v, idxd_v, bufh, bufb):
    c = lax.axis_index("c")
    s = lax.axis_index("s")
    wid = s * NC + c
    pltpu.sync_copy(srcg_hbm.at[wid], idxs_v)
    pltpu.sync_copy(dstg_hbm.at[wid], idxd_v)
    base = wid * TG

    def chunk(i, carry):
        off = base + i * G1C
        pltpu.sync_copy(haug_hbm.at[idxs_v.at[i]], bufh)
        pltpu.sync_copy(bufh, hsa_hbm.at[pl.ds(off, G1C), :])
        pltpu.sync_copy(ad_hbm.at[idxd_v.at[i]], bufb)
        pltpu.sync_copy(bufb, adg_hbm.at[pl.ds(off, G1C), :])
        return carry

    lax.fori_loop(0, G1N, chunk, 0)


def _sc1(haug, ad128, srcg3, dstg3):
    mesh = plsc.VectorSubcoreMesh(
        core_axis_name="c", subcore_axis_name="s",
        num_cores=NC, num_subcores=NS)
    return pl.kernel(
        _sc1_body,
        out_type=[
            jax.ShapeDtypeStruct((EGP, NH * CH + CH), jnp.float32),
            jax.ShapeDtypeStruct((EGP, CH), jnp.float32),
        ],
        mesh=mesh,
        scratch_types=[
            pltpu.VMEM((G1N, G1C), jnp.int32),
            pltpu.VMEM((G1N, G1C), jnp.int32),
            pltpu.VMEM((G1C, NH * CH + CH), jnp.float32),
            pltpu.VMEM((G1C, CH), jnp.float32),
        ],
    )(haug, ad128, srcg3, dstg3)


def _tc2_body(hsa_ref, adg_ref, w_ref, msg_ref):
    i = pl.program_id(0)
    hsa = hsa_ref[...]
    a = hsa[:, NH * CH:NH * CH + NH] + adg_ref[...][:, :NH]
    a = jnp.where(a >= 0.0, a, 0.2 * a)
    w = jnp.exp(a)
    eid = i * BE + lax.broadcasted_iota(jnp.int32, (BE, 1), 0)
    w = jnp.where(eid < EG, w, 0.0)
    w_ref[...] = jnp.concatenate([w, jnp.zeros((BE, 124), jnp.float32)], axis=1)
    for q in range(NH):
        msg_ref[q, :, :] = hsa[:, q * CH:(q + 1) * CH] * w[:, q:q + 1]


def _tc2(hsa, adg):
    return pl.pallas_call(
        _tc2_body,
        grid=(EGP // BE,),
        in_specs=[
            pl.BlockSpec((BE, NH * CH + CH), lambda i: (i, 0)),
            pl.BlockSpec((BE, CH), lambda i: (i, 0)),
        ],
        out_specs=[
            pl.BlockSpec((BE, CH), lambda i: (i, 0)),
            pl.BlockSpec((NH, BE, CH), lambda i: (0, i, 0)),
        ],
        out_shape=[
            jax.ShapeDtypeStruct((EGP, CH), jnp.float32),
            jax.ShapeDtypeStruct((NH, EGP, CH), jnp.float32),
        ],
    )(hsa, adg)


def _sc2_body(msg_hbm, w_hbm, dsts_hbm, z128_hbm,
              nump_hbm, denp_hbm,
              spA, idx_v, buf):
    c = lax.axis_index("c")
    s = lax.axis_index("s")
    wid = s * NC + c
    pltpu.sync_copy(dsts_hbm.at[wid], idx_v)
    base = wid * TG

    pltpu.sync_copy(z128_hbm, spA.at[pl.ds(s * ZB, ZB)])
    plsc.subcore_barrier()

    def dchunk(i, carry):
        pltpu.sync_copy(w_hbm.at[pl.ds(base + i * S2C, S2C), :], buf)
        pltpu.sync_copy(buf, spA.at[idx_v.at[i]], add=True)
        return carry

    lax.fori_loop(0, S2N, dchunk, 0)
    plsc.subcore_barrier()
    pltpu.sync_copy(spA.at[pl.ds(s * CB, CB)],
                    denp_hbm.at[c, pl.ds(s * CB, CB), :])

    @pl.when(s == 0)
    def _():
        pltpu.sync_copy(spA.at[pl.ds(NS * CB, CBT)],
                        denp_hbm.at[c, pl.ds(NS * CB, CBT), :])

    for q in range(NH):
        plsc.subcore_barrier()
        pltpu.sync_copy(z128_hbm, spA.at[pl.ds(s * ZB, ZB)])
        plsc.subcore_barrier()

        def chunk(i, carry):
            pltpu.sync_copy(msg_hbm.at[q, pl.ds(base + i * S2C, S2C), :], buf)
            pltpu.sync_copy(buf, spA.at[idx_v.at[i]], add=True)
            return carry

        lax.fori_loop(0, S2N, chunk, 0)
        plsc.subcore_barrier()
        pltpu.sync_copy(spA.at[pl.ds(s * CB, CB)],
                        nump_hbm.at[c, q, pl.ds(s * CB, CB), :])

        @pl.when(s == 0)
        def _():
            pltpu.sync_copy(spA.at[pl.ds(NS * CB, CBT)],
                            nump_hbm.at[c, q, pl.ds(NS * CB, CBT), :])


def _sc2(msg4, w128, dsts3, z128):
    mesh = plsc.VectorSubcoreMesh(
        core_axis_name="c", subcore_axis_name="s",
        num_cores=NC, num_subcores=NS)
    return pl.kernel(
        _sc2_body,
        out_type=[
            jax.ShapeDtypeStruct((NC, NH, NN, CH), jnp.float32),
            jax.ShapeDtypeStruct((NC, NN, CH), jnp.float32),
        ],
        mesh=mesh,
        scratch_types=[
            pltpu.VMEM_SHARED((SR, CH), jnp.float32),
            pltpu.VMEM((S2N, S2C), jnp.int32),
            pltpu.VMEM((S2C, CH), jnp.float32),
        ],
    )(msg4, w128, dsts3, z128)


def _tc3_body(nump_ref, denp_ref, bg_ref, out4_ref):
    num = nump_ref[...]
    den = denp_ref[...]
    ns = num[0] + num[1]
    d = den[0][:, :NH] + den[1][:, :NH]
    bg = bg_ref[...]
    for q in range(NH):
        o = ns[q] / d[:, q:q + 1] + bg[q][None, :]
        out4_ref[q, :, :] = jnp.maximum(o, 0.0)


def _tc3(nump, denp, bg4):
    return pl.pallas_call(
        _tc3_body,
        grid=(NN // BN,),
        in_specs=[
            pl.BlockSpec((NC, NH, BN, CH), lambda i: (0, 0, i, 0)),
            pl.BlockSpec((NC, BN, CH), lambda i: (0, i, 0)),
            pl.BlockSpec((NH, CH), lambda i: (0, 0)),
        ],
        out_specs=pl.BlockSpec((NH, BN, CH), lambda i: (0, i, 0)),
        out_shape=jax.ShapeDtypeStruct((NH, NN, CH), jnp.float32),
    )(nump, denp, bg4)


def _sc3_body(out4_hbm, srcn_hbm, dstn_hbm, z128_hbm,
              aggp_hbm,
              spA, idxs_v, idxd_v, buf):
    c = lax.axis_index("c")
    s = lax.axis_index("s")
    wid = s * NC + c
    pltpu.sync_copy(srcn_hbm.at[wid], idxs_v)
    pltpu.sync_copy(dstn_hbm.at[wid], idxd_v)

    for q in range(NH):
        plsc.subcore_barrier()
        pltpu.sync_copy(z128_hbm, spA.at[pl.ds(s * ZB, ZB)])
        plsc.subcore_barrier()

        def chunk(i, carry):
            pltpu.sync_copy(out4_hbm.at[q].at[idxs_v.at[i]], buf)
            pltpu.sync_copy(buf, spA.at[idxd_v.at[i]], add=True)
            return carry

        lax.fori_loop(0, S3N, chunk, 0)
        plsc.subcore_barrier()
        pltpu.sync_copy(spA.at[pl.ds(s * CB, CB)],
                        aggp_hbm.at[c, q, pl.ds(s * CB, CB), :])

        @pl.when(s == 0)
        def _():
            pltpu.sync_copy(spA.at[pl.ds(NS * CB, CBT)],
                            aggp_hbm.at[c, q, pl.ds(NS * CB, CBT), :])


def _sc3(out4, srcn3, dstn3, z128):
    mesh = plsc.VectorSubcoreMesh(
        core_axis_name="c", subcore_axis_name="s",
        num_cores=NC, num_subcores=NS)
    return pl.kernel(
        _sc3_body,
        out_type=jax.ShapeDtypeStruct((NC, NH, NN, CH), jnp.float32),
        mesh=mesh,
        scratch_types=[
            pltpu.VMEM_SHARED((SR, CH), jnp.float32),
            pltpu.VMEM((S3N, S3C), jnp.int32),
            pltpu.VMEM((S3N, S3C), jnp.int32),
            pltpu.VMEM((S3C, CH), jnp.float32),
        ],
    )(out4, srcn3, dstn3, z128)


def _tc4_body(out4_ref, aggp_ref, w1_ref, b1_ref, w2_ref, b2_ref,
              batch_ref, wf_ref, bf_ref, res_ref, psum):
    i = pl.program_id(0)

    @pl.when(i == 0)
    def _():
        psum[...] = jnp.zeros((NG, 128), jnp.float32)

    agg = aggp_ref[...]
    z = out4_ref[...] + agg[0] + agg[1]
    acc = jnp.zeros((BN, 128), jnp.float32)
    w1 = w1_ref[...]
    for q in range(NH):
        acc = acc + jnp.dot(z[q], w1[q], preferred_element_type=jnp.float32)
    y1 = jnp.maximum(acc + b1_ref[...], 0.0)
    y2 = jnp.dot(y1, w2_ref[...], preferred_element_type=jnp.float32) + b2_ref[...]
    y2aug = jnp.concatenate(
        [y2, jnp.ones((BN, 1), jnp.float32), jnp.zeros((BN, 63), jnp.float32)],
        axis=1)
    b = batch_ref[...][0, 0, :]
    gid = lax.broadcasted_iota(jnp.int32, (NG, BN), 0)
    onehot_t = (gid == b[None, :]).astype(jnp.float32)
    psum[...] += jnp.dot(onehot_t, y2aug, preferred_element_type=jnp.float32)

    @pl.when(i == NN // BN - 1)
    def _():
        p = psum[...]
        pooled = p[:, :NG] / jnp.maximum(p[:, NG:NG + 1], 1.0)
        res_ref[...] = jnp.dot(pooled, wf_ref[...],
                               preferred_element_type=jnp.float32) + bf_ref[...]


def _tc4(out4, aggp, w1_4, b1r, w2, b2r, batch3, wf, bfr):
    return pl.pallas_call(
        _tc4_body,
        grid=(NN // BN,),
        in_specs=[
            pl.BlockSpec((NH, BN, CH), lambda i: (0, i, 0)),
            pl.BlockSpec((NC, NH, BN, CH), lambda i: (0, 0, i, 0)),
            pl.BlockSpec((NH, CH, 128), lambda i: (0, 0, 0)),
            pl.BlockSpec((1, 128), lambda i: (0, 0)),
            pl.BlockSpec((128, NG), lambda i: (0, 0)),
            pl.BlockSpec((1, NG), lambda i: (0, 0)),
            pl.BlockSpec((1, 1, BN), lambda i: (i, 0, 0)),
            pl.BlockSpec((NG, 1), lambda i: (0, 0)),
            pl.BlockSpec((1, 1), lambda i: (0, 0)),
        ],
        out_specs=pl.BlockSpec((NG, 1), lambda i: (0, 0)),
        out_shape=jax.ShapeDtypeStruct((NG, 1), jnp.float32),
        scratch_shapes=[pltpu.VMEM((NG, 128), jnp.float32)],
    )(out4, aggp, w1_4, b1r, w2, b2r, batch3, wf, bfr)


def kernel(x, edge_index, batch, W_gat, att_src, att_dst, b_gat,
           W1, b1, W2, b2, Wf, bf):
    src0 = edge_index[0].astype(jnp.int32)
    dst0 = edge_index[1].astype(jnp.int32)
    loop = jnp.arange(NN, dtype=jnp.int32)
    padg = jnp.zeros((EGP - EG,), jnp.int32)
    srcg = jnp.concatenate([src0, loop, padg])
    dstg = jnp.concatenate([dst0, loop, padg])
    srcg3 = srcg.reshape(NW, G1N, G1C)
    dstg3 = dstg.reshape(NW, G1N, G1C)
    dsts3 = dstg.reshape(NW, S2N, S2C)
    srcn = jnp.concatenate([src0, jnp.zeros((ENP - EE,), jnp.int32)])
    dstn = jnp.concatenate(
        [dst0, jnp.full((ENP - EE,), NN, jnp.int32)])
    srcn3 = srcn.reshape(NW, S3N, S3C)
    dstn3 = dstn.reshape(NW, S3N, S3C)

    z128 = jnp.zeros((ZB, CH), jnp.float32)

    haug, ad128 = _tc1(x, W_gat, att_src, att_dst)
    hsa, adg = _sc1(haug, ad128, srcg3, dstg3)
    w128, msg4 = _tc2(hsa, adg)
    nump, denp = _sc2(msg4, w128, dsts3, z128)
    out4 = _tc3(nump, denp, b_gat.reshape(NH, CH))
    aggp = _sc3(out4, srcn3, dstn3, z128)
    res = _tc4(out4, aggp, W1.reshape(NH, CH, 128), b1.reshape(1, 128),
               W2, b2.reshape(1, NG), batch.astype(jnp.int32).reshape(NN // BN, 1, BN),
               Wf, bf.reshape(1, 1))
    return res


# speedup vs baseline: 10.1832x; 1.1397x over previous
import functools

import jax
import jax.numpy as jnp
from jax import lax
from jax.experimental import pallas as pl
from jax.experimental.pallas import tpu as pltpu
from jax.experimental.pallas import tpu_sc as plsc

NN = 10000
EE = 160000
DIN = 128
NH = 4
CH = 128
NG = 64

NC = 2
NS = 16
NW = NC * NS

EG = EE + NN
EGP = 172032
TG = EGP // NW
G1C = 64
G1N = TG // G1C
S2C = 128
S2N = TG // S2C

ENP = 163840
TN = ENP // NW
S3C = 128
S3N = TN // S3C

SR = 10240
ZB = 640
CB = 624
CBT = 16

BN = 1000
BE = 1024


def _tc1_body(x_ref, w_ref, atts_ref, attd_ref, haug_ref, ad_ref):
    h = jnp.dot(x_ref[...], w_ref[...], preferred_element_type=jnp.float32)
    hr = h.reshape(BN, NH, CH)
    a_s = jnp.sum(hr * atts_ref[...][None], axis=-1)
    a_d = jnp.sum(hr * attd_ref[...][None], axis=-1)
    z = jnp.zeros((BN, 124), jnp.float32)
    haug_ref[...] = jnp.concatenate([h, a_s, z], axis=1)
    ad_ref[...] = jnp.concatenate([a_d, z], axis=1)


def _tc1(x, w_gat, att_src, att_dst):
    return pl.pallas_call(
        _tc1_body,
        grid=(NN // BN,),
        in_specs=[
            pl.BlockSpec((BN, DIN), lambda i: (i, 0)),
            pl.BlockSpec((DIN, NH * CH), lambda i: (0, 0)),
            pl.BlockSpec((NH, CH), lambda i: (0, 0)),
            pl.BlockSpec((NH, CH), lambda i: (0, 0)),
        ],
        out_specs=[
            pl.BlockSpec((BN, NH * CH + CH), lambda i: (i, 0)),
            pl.BlockSpec((BN, CH), lambda i: (i, 0)),
        ],
        out_shape=[
            jax.ShapeDtypeStruct((NN, NH * CH + CH), jnp.float32),
            jax.ShapeDtypeStruct((NN, CH), jnp.float32),
        ],
    )(x, w_gat, att_src, att_dst)


def _sc1_body(haug_hbm, ad_hbm, srcg_hbm, dstg_hbm,
              hsa_hbm, adg_hbm,
              idxs_v, idxd_v, bufh0, bufh1, bufb0, bufb1,
              sgh0, sgh1, sgb0, sgb1, swh0, swh1, swb0, swb1):
    c = lax.axis_index("c")
    s = lax.axis_index("s")
    wid = s * NC + c
    pltpu.sync_copy(srcg_hbm.at[wid], idxs_v)
    pltpu.sync_copy(dstg_hbm.at[wid], idxd_v)
    base = wid * TG
    bufh = (bufh0, bufh1)
    bufb = (bufb0, bufb1)
    sgh = (sgh0, sgh1)
    sgb = (sgb0, sgb1)
    swh = (swh0, swh1)
    swb = (swb0, swb1)

    def issue_gather(i, p):
        pltpu.async_copy(haug_hbm.at[idxs_v.at[i]], bufh[p], sgh[p])
        pltpu.async_copy(ad_hbm.at[idxd_v.at[i]], bufb[p], sgb[p])

    def wait_gather(p):
        pltpu.make_async_copy(haug_hbm.at[idxs_v.at[0]], bufh[p], sgh[p]).wait()
        pltpu.make_async_copy(ad_hbm.at[idxd_v.at[0]], bufb[p], sgb[p]).wait()

    def issue_write(i, p):
        off = base + i * G1C
        pltpu.async_copy(bufh[p], hsa_hbm.at[pl.ds(off, G1C), :], swh[p])
        pltpu.async_copy(bufb[p], adg_hbm.at[pl.ds(off, G1C), :], swb[p])

    def wait_write(p):
        pltpu.make_async_copy(bufh[p], hsa_hbm.at[pl.ds(base, G1C), :], swh[p]).wait()
        pltpu.make_async_copy(bufb[p], adg_hbm.at[pl.ds(base, G1C), :], swb[p]).wait()

    issue_gather(0, 0)

    def body(j, carry):
        i0 = 2 * j
        wait_gather(0)
        issue_write(i0, 0)
        issue_gather(i0 + 1, 1)
        wait_gather(1)
        issue_write(i0 + 1, 1)
        wait_write(0)
        issue_gather(jnp.minimum(i0 + 2, G1N - 1), 0)
        wait_write(1)
        return carry

    lax.fori_loop(0, G1N // 2, body, 0)
    wait_gather(0)


def _sc1(haug, ad128, srcg3, dstg3):
    mesh = plsc.VectorSubcoreMesh(
        core_axis_name="c", subcore_axis_name="s",
        num_cores=NC, num_subcores=NS)
    return pl.kernel(
        _sc1_body,
        out_type=[
            jax.ShapeDtypeStruct((EGP, NH * CH + CH), jnp.float32),
            jax.ShapeDtypeStruct((EGP, CH), jnp.float32),
        ],
        mesh=mesh,
        scratch_types=[
            pltpu.VMEM((G1N, G1C), jnp.int32),
            pltpu.VMEM((G1N, G1C), jnp.int32),
            pltpu.VMEM((G1C, NH * CH + CH), jnp.float32),
            pltpu.VMEM((G1C, NH * CH + CH), jnp.float32),
            pltpu.VMEM((G1C, CH), jnp.float32),
            pltpu.VMEM((G1C, CH), jnp.float32),
        ] + [pltpu.SemaphoreType.DMA] * 8,
    )(haug, ad128, srcg3, dstg3)


def _tc2_body(hsa_ref, adg_ref, w_ref, msg_ref):
    i = pl.program_id(0)
    hsa = hsa_ref[...]
    a = hsa[:, NH * CH:NH * CH + NH] + adg_ref[...][:, :NH]
    a = jnp.where(a >= 0.0, a, 0.2 * a)
    w = jnp.exp(a)
    eid = i * BE + lax.broadcasted_iota(jnp.int32, (BE, 1), 0)
    w = jnp.where(eid < EG, w, 0.0)
    w_ref[...] = jnp.concatenate([w, jnp.zeros((BE, 124), jnp.float32)], axis=1)
    for q in range(NH):
        msg_ref[q, :, :] = hsa[:, q * CH:(q + 1) * CH] * w[:, q:q + 1]


def _tc2(hsa, adg):
    return pl.pallas_call(
        _tc2_body,
        grid=(EGP // BE,),
        in_specs=[
            pl.BlockSpec((BE, NH * CH + CH), lambda i: (i, 0)),
            pl.BlockSpec((BE, CH), lambda i: (i, 0)),
        ],
        out_specs=[
            pl.BlockSpec((BE, CH), lambda i: (i, 0)),
            pl.BlockSpec((NH, BE, CH), lambda i: (0, i, 0)),
        ],
        out_shape=[
            jax.ShapeDtypeStruct((EGP, CH), jnp.float32),
            jax.ShapeDtypeStruct((NH, EGP, CH), jnp.float32),
        ],
    )(hsa, adg)


def _scatter_pass(read_at, n, spA, idx_v, buf0, buf1, sr0, sr1, ss0, ss1):
    buf = (buf0, buf1)
    sr = (sr0, sr1)
    ss = (ss0, ss1)

    def issue_read(i, p):
        pltpu.async_copy(read_at(i), buf[p], sr[p])

    def wait_read(p):
        pltpu.make_async_copy(read_at(0), buf[p], sr[p]).wait()

    def issue_scat(i, p):
        pltpu.async_copy(buf[p], spA.at[idx_v.at[i]], ss[p], add=True)

    def wait_scat(p):
        pltpu.make_async_copy(buf[p], spA.at[idx_v.at[0]], ss[p]).wait()

    issue_read(0, 0)

    def body(j, carry):
        i0 = 2 * j
        wait_read(0)
        issue_scat(i0, 0)
        issue_read(i0 + 1, 1)
        wait_read(1)
        issue_scat(i0 + 1, 1)
        wait_scat(0)
        issue_read(jnp.minimum(i0 + 2, n - 1), 0)
        wait_scat(1)
        return carry

    lax.fori_loop(0, n // 2, body, 0)
    wait_read(0)


def _sc2_body(msg_hbm, w_hbm, dsts_hbm, z128_hbm,
              nump_hbm, denp_hbm,
              spA, idx_v, buf0, buf1, sr0, sr1, ss0, ss1):
    c = lax.axis_index("c")
    s = lax.axis_index("s")
    wid = s * NC + c
    pltpu.sync_copy(dsts_hbm.at[wid], idx_v)
    base = wid * TG

    pltpu.sync_copy(z128_hbm, spA.at[pl.ds(s * ZB, ZB)])
    plsc.subcore_barrier()
    _scatter_pass(lambda i: w_hbm.at[pl.ds(base + i * S2C, S2C), :],
                  S2N, spA, idx_v, buf0, buf1, sr0, sr1, ss0, ss1)
    plsc.subcore_barrier()
    pltpu.sync_copy(spA.at[pl.ds(s * CB, CB)],
                    denp_hbm.at[c, pl.ds(s * CB, CB), :])

    @pl.when(s == 0)
    def _():
        pltpu.sync_copy(spA.at[pl.ds(NS * CB, CBT)],
                        denp_hbm.at[c, pl.ds(NS * CB, CBT), :])

    for q in range(NH):
        plsc.subcore_barrier()
        pltpu.sync_copy(z128_hbm, spA.at[pl.ds(s * ZB, ZB)])
        plsc.subcore_barrier()
        _scatter_pass(lambda i: msg_hbm.at[q, pl.ds(base + i * S2C, S2C), :],
                      S2N, spA, idx_v, buf0, buf1, sr0, sr1, ss0, ss1)
        plsc.subcore_barrier()
        pltpu.sync_copy(spA.at[pl.ds(s * CB, CB)],
                        nump_hbm.at[c, q, pl.ds(s * CB, CB), :])

        @pl.when(s == 0)
        def _():
            pltpu.sync_copy(spA.at[pl.ds(NS * CB, CBT)],
                            nump_hbm.at[c, q, pl.ds(NS * CB, CBT), :])


def _sc2(msg4, w128, dsts3, z128):
    mesh = plsc.VectorSubcoreMesh(
        core_axis_name="c", subcore_axis_name="s",
        num_cores=NC, num_subcores=NS)
    return pl.kernel(
        _sc2_body,
        out_type=[
            jax.ShapeDtypeStruct((NC, NH, NN, CH), jnp.float32),
            jax.ShapeDtypeStruct((NC, NN, CH), jnp.float32),
        ],
        mesh=mesh,
        scratch_types=[
            pltpu.VMEM_SHARED((SR, CH), jnp.float32),
            pltpu.VMEM((S2N, S2C), jnp.int32),
            pltpu.VMEM((S2C, CH), jnp.float32),
            pltpu.VMEM((S2C, CH), jnp.float32),
        ] + [pltpu.SemaphoreType.DMA] * 4,
    )(msg4, w128, dsts3, z128)


def _tc3_body(nump_ref, denp_ref, bg_ref, out4_ref):
    num = nump_ref[...]
    den = denp_ref[...]
    ns = num[0] + num[1]
    d = den[0][:, :NH] + den[1][:, :NH]
    bg = bg_ref[...]
    for q in range(NH):
        o = ns[q] / d[:, q:q + 1] + bg[q][None, :]
        out4_ref[q, :, :] = jnp.maximum(o, 0.0)


def _tc3(nump, denp, bg4):
    return pl.pallas_call(
        _tc3_body,
        grid=(NN // BN,),
        in_specs=[
            pl.BlockSpec((NC, NH, BN, CH), lambda i: (0, 0, i, 0)),
            pl.BlockSpec((NC, BN, CH), lambda i: (0, i, 0)),
            pl.BlockSpec((NH, CH), lambda i: (0, 0)),
        ],
        out_specs=pl.BlockSpec((NH, BN, CH), lambda i: (0, i, 0)),
        out_shape=jax.ShapeDtypeStruct((NH, NN, CH), jnp.float32),
    )(nump, denp, bg4)


def _sc3_body(out4_hbm, srcn_hbm, dstn_hbm, z128_hbm,
              aggp_hbm,
              spA, idxs_v, idxd_v, buf0, buf1, sr0, sr1, ss0, ss1):
    c = lax.axis_index("c")
    s = lax.axis_index("s")
    wid = s * NC + c
    pltpu.sync_copy(srcn_hbm.at[wid], idxs_v)
    pltpu.sync_copy(dstn_hbm.at[wid], idxd_v)

    for q in range(NH):
        plsc.subcore_barrier()
        pltpu.sync_copy(z128_hbm, spA.at[pl.ds(s * ZB, ZB)])
        plsc.subcore_barrier()
        _scatter_pass(lambda i: out4_hbm.at[q].at[idxs_v.at[i]],
                      S3N, spA, idxd_v, buf0, buf1, sr0, sr1, ss0, ss1)
        plsc.subcore_barrier()
        pltpu.sync_copy(spA.at[pl.ds(s * CB, CB)],
                        aggp_hbm.at[c, q, pl.ds(s * CB, CB), :])

        @pl.when(s == 0)
        def _():
            pltpu.sync_copy(spA.at[pl.ds(NS * CB, CBT)],
                            aggp_hbm.at[c, q, pl.ds(NS * CB, CBT), :])


def _sc3(out4, srcn3, dstn3, z128):
    mesh = plsc.VectorSubcoreMesh(
        core_axis_name="c", subcore_axis_name="s",
        num_cores=NC, num_subcores=NS)
    return pl.kernel(
        _sc3_body,
        out_type=jax.ShapeDtypeStruct((NC, NH, NN, CH), jnp.float32),
        mesh=mesh,
        scratch_types=[
            pltpu.VMEM_SHARED((SR, CH), jnp.float32),
            pltpu.VMEM((S3N, S3C), jnp.int32),
            pltpu.VMEM((S3N, S3C), jnp.int32),
            pltpu.VMEM((S3C, CH), jnp.float32),
            pltpu.VMEM((S3C, CH), jnp.float32),
        ] + [pltpu.SemaphoreType.DMA] * 4,
    )(out4, srcn3, dstn3, z128)


def _tc4_body(out4_ref, aggp_ref, w1_ref, b1_ref, w2_ref, b2_ref,
              batch_ref, wf_ref, bf_ref, res_ref, psum):
    i = pl.program_id(0)

    @pl.when(i == 0)
    def _():
        psum[...] = jnp.zeros((NG, 128), jnp.float32)

    agg = aggp_ref[...]
    z = out4_ref[...] + agg[0] + agg[1]
    acc = jnp.zeros((BN, 128), jnp.float32)
    w1 = w1_ref[...]
    for q in range(NH):
        acc = acc + jnp.dot(z[q], w1[q], preferred_element_type=jnp.float32)
    y1 = jnp.maximum(acc + b1_ref[...], 0.0)
    y2 = jnp.dot(y1, w2_ref[...], preferred_element_type=jnp.float32) + b2_ref[...]
    y2aug = jnp.concatenate(
        [y2, jnp.ones((BN, 1), jnp.float32), jnp.zeros((BN, 63), jnp.float32)],
        axis=1)
    b = batch_ref[...][0, 0, :]
    gid = lax.broadcasted_iota(jnp.int32, (NG, BN), 0)
    onehot_t = (gid == b[None, :]).astype(jnp.float32)
    psum[...] += jnp.dot(onehot_t, y2aug, preferred_element_type=jnp.float32)

    @pl.when(i == NN // BN - 1)
    def _():
        p = psum[...]
        pooled = p[:, :NG] / jnp.maximum(p[:, NG:NG + 1], 1.0)
        res_ref[...] = jnp.dot(pooled, wf_ref[...],
                               preferred_element_type=jnp.float32) + bf_ref[...]


def _tc4(out4, aggp, w1_4, b1r, w2, b2r, batch3, wf, bfr):
    return pl.pallas_call(
        _tc4_body,
        grid=(NN // BN,),
        in_specs=[
            pl.BlockSpec((NH, BN, CH), lambda i: (0, i, 0)),
            pl.BlockSpec((NC, NH, BN, CH), lambda i: (0, 0, i, 0)),
            pl.BlockSpec((NH, CH, 128), lambda i: (0, 0, 0)),
            pl.BlockSpec((1, 128), lambda i: (0, 0)),
            pl.BlockSpec((128, NG), lambda i: (0, 0)),
            pl.BlockSpec((1, NG), lambda i: (0, 0)),
            pl.BlockSpec((1, 1, BN), lambda i: (i, 0, 0)),
            pl.BlockSpec((NG, 1), lambda i: (0, 0)),
            pl.BlockSpec((1, 1), lambda i: (0, 0)),
        ],
        out_specs=pl.BlockSpec((NG, 1), lambda i: (0, 0)),
        out_shape=jax.ShapeDtypeStruct((NG, 1), jnp.float32),
        scratch_shapes=[pltpu.VMEM((NG, 128), jnp.float32)],
    )(out4, aggp, w1_4, b1r, w2, b2r, batch3, wf, bfr)


def kernel(x, edge_index, batch, W_gat, att_src, att_dst, b_gat,
           W1, b1, W2, b2, Wf, bf):
    src0 = edge_index[0].astype(jnp.int32)
    dst0 = edge_index[1].astype(jnp.int32)
    loop = jnp.arange(NN, dtype=jnp.int32)
    padg = jnp.zeros((EGP - EG,), jnp.int32)
    srcg = jnp.concatenate([src0, loop, padg])
    dstg = jnp.concatenate([dst0, loop, padg])
    srcg3 = srcg.reshape(NW, G1N, G1C)
    dstg3 = dstg.reshape(NW, G1N, G1C)
    dsts3 = dstg.reshape(NW, S2N, S2C)
    srcn = jnp.concatenate([src0, jnp.zeros((ENP - EE,), jnp.int32)])
    dstn = jnp.concatenate(
        [dst0, jnp.full((ENP - EE,), NN, jnp.int32)])
    srcn3 = srcn.reshape(NW, S3N, S3C)
    dstn3 = dstn.reshape(NW, S3N, S3C)

    z128 = jnp.zeros((ZB, CH), jnp.float32)

    haug, ad128 = _tc1(x, W_gat, att_src, att_dst)
    hsa, adg = _sc1(haug, ad128, srcg3, dstg3)
    w128, msg4 = _tc2(hsa, adg)
    nump, denp = _sc2(msg4, w128, dsts3, z128)
    out4 = _tc3(nump, denp, b_gat.reshape(NH, CH))
    aggp = _sc3(out4, srcn3, dstn3, z128)
    res = _tc4(out4, aggp, W1.reshape(NH, CH, 128), b1.reshape(1, 128),
               W2, b2.reshape(1, NG), batch.astype(jnp.int32).reshape(NN // BN, 1, BN),
               Wf, bf.reshape(1, 1))
    return res
